# baseline (speedup 1.0000x reference)
import functools

import jax
import jax.numpy as jnp
from jax import lax
from jax.experimental import pallas as pl
from jax.experimental.pallas import tpu as pltpu
from jax.experimental.pallas import tpu_sc as plsc

N_TILES = 32
EDGE_BLK = 64
N_BLK = 160
CHUNK = 16
NBUF = 4
N_CHUNK = N_BLK // CHUNK
EPT = EDGE_BLK * N_BLK
D = 128
L = 16
R_PAD = 10016
ROWS_PT = R_PAD // 16


def _sc_aggregate(x, srcp, dstp, zrow):
  mesh = plsc.VectorSubcoreMesh(core_axis_name="c", subcore_axis_name="s")

  @functools.partial(
      pl.kernel,
      out_type=[
          jax.ShapeDtypeStruct((2, R_PAD, D), jnp.float32),
          jax.ShapeDtypeStruct((N_TILES, R_PAD), jnp.float32),
      ],
      mesh=mesh,
      compiler_params=pltpu.CompilerParams(
          use_tc_tiling_on_sc=False, needs_layout_passes=False),
      scratch_types=[
          pltpu.VMEM((CHUNK, EDGE_BLK), jnp.int32),
          pltpu.VMEM((CHUNK, EDGE_BLK), jnp.int32),
          [pltpu.VMEM((EDGE_BLK, D), jnp.float32)] * NBUF,
          pltpu.VMEM((R_PAD,), jnp.float32),
          pltpu.VMEM_SHARED((R_PAD, D), jnp.float32),
          [pltpu.SemaphoreType.DMA] * NBUF,
          [pltpu.SemaphoreType.DMA] * NBUF,
      ],
  )
  def k(x_hbm, src_hbm, dst_hbm, zrow_hbm, psum_hbm, pcnt_hbm,
        srcv, dstv, rbufs, hist, accum, sg, ss):
    cid = lax.axis_index("c")
    sid = lax.axis_index("s")
    wid = cid * 16 + sid
    base = sid * ROWS_PT

    ones_v = jnp.ones((L,), jnp.float32)
    zero_v = jnp.zeros((L,), jnp.float32)

    def init_hist(r, carry):
      hist[pl.ds(r * L, L)] = zero_v
      return carry

    lax.fori_loop(0, R_PAD // L, init_hist, 0)

    pltpu.sync_copy(zrow_hbm, accum.at[pl.ds(base, ROWS_PT)])
    plsc.subcore_barrier()

    def fire_gather(j, b):
      pltpu.async_copy(x_hbm.at[srcv.at[j]], rbufs[b], sg[b])

    def wait_gather(j, b):
      pltpu.make_async_copy(x_hbm.at[srcv.at[j]], rbufs[b], sg[b]).wait()

    def fire_scat(j, b):
      pltpu.async_copy(rbufs[b], accum.at[dstv.at[j]], ss[b], add=True)
      for u in range(EDGE_BLK // L):
        idx = dstv[j, pl.ds(u * L, L)]
        plsc.addupdate_scatter(hist, [idx], ones_v)

    def wait_scat(j, b):
      pltpu.make_async_copy(rbufs[b], accum.at[dstv.at[j]], ss[b]).wait()

    G = CHUNK // NBUF

    def chunk_body(c, carry):
      pltpu.sync_copy(src_hbm.at[wid, pl.ds(c * CHUNK, CHUNK)], srcv)
      pltpu.sync_copy(dst_hbm.at[wid, pl.ds(c * CHUNK, CHUNK)], dstv)
      for b in range(NBUF):
        fire_gather(b, b)

      def grp(t, inner):
        j0 = NBUF * t
        for b in range(NBUF):
          wait_gather(j0 + b, b)
          fire_scat(j0 + b, b)
        for b in range(NBUF):
          wait_scat(j0 + b, b)
          fire_gather(j0 + NBUF + b, b)
        return inner

      lax.fori_loop(0, G - 1, grp, carry)
      j0 = NBUF * (G - 1)
      for b in range(NBUF):
        wait_gather(j0 + b, b)
        fire_scat(j0 + b, b)
      for b in range(NBUF):
        wait_scat(j0 + b, b)
      return carry

    lax.fori_loop(0, N_CHUNK, chunk_body, 0)
    plsc.subcore_barrier()
    pltpu.sync_copy(accum.at[pl.ds(base, ROWS_PT)],
                    psum_hbm.at[cid, pl.ds(base, ROWS_PT)])
    pltpu.sync_copy(hist, pcnt_hbm.at[wid])

  return k(x, srcp, dstp, zrow)


def _normalize(psum, pcnt, b):
  blk = 1000

  def body(ps_ref, pc_ref, o_ref):
    s = ps_ref[0] + ps_ref[1]
    c = jnp.sum(pc_ref[...], axis=1)
    o_ref[...] = s / jnp.maximum(c, 1.0)[:, None]

  return pl.pallas_call(
      body,
      grid=(b // blk,),
      in_specs=[
          pl.BlockSpec((2, blk, D), lambda i: (0, i, 0)),
          pl.BlockSpec((blk, N_TILES), lambda i: (i, 0)),
      ],
      out_specs=pl.BlockSpec((blk, D), lambda i: (i, 0)),
      out_shape=jax.ShapeDtypeStruct((b, D), jnp.float32),
  )(psum[:, :b], pcnt[:b])


def kernel(x, nodes, edge_index):
  b = nodes.shape[0]
  n = x.shape[0]
  e = edge_index.shape[1]
  pad = N_TILES * EPT - e
  ppt = pad // N_TILES
  pad_src = jnp.broadcast_to(
      (jnp.arange(ppt, dtype=jnp.int32) * 41) % n, (N_TILES, ppt))
  pad_dst = jnp.broadcast_to(
      b + (jnp.arange(ppt, dtype=jnp.int32) % (R_PAD - b)), (N_TILES, ppt))
  src = jnp.concatenate([edge_index[0].reshape(N_TILES, -1), pad_src], axis=1)
  dst = jnp.concatenate([edge_index[1].reshape(N_TILES, -1), pad_dst], axis=1)
  srcp = src.reshape(N_TILES, N_BLK, EDGE_BLK)
  dstp = dst.reshape(N_TILES, N_BLK, EDGE_BLK)
  zrow = jnp.zeros((ROWS_PT, D), jnp.float32)
  psum, pcnt = _sc_aggregate(x, srcp, dstp, zrow)
  return _normalize(psum, pcnt.T, b)


# speedup vs baseline: 1.0142x; 1.0142x over previous
import functools

import jax
import jax.numpy as jnp
from jax import lax
from jax.experimental import pallas as pl
from jax.experimental.pallas import tpu as pltpu
from jax.experimental.pallas import tpu_sc as plsc

N_TILES = 32
EDGE_BLK = 128
N_BLK = 80
CHUNK = 8
N_CHUNK = N_BLK // CHUNK
EPT = EDGE_BLK * N_BLK
D = 128
L = 16
R_PAD = 10112
ROWS_PT = R_PAD // 16


def _sc_aggregate(x, srcp, dstp, zrow):
  mesh = plsc.VectorSubcoreMesh(core_axis_name="c", subcore_axis_name="s")

  @functools.partial(
      pl.kernel,
      out_type=[
          jax.ShapeDtypeStruct((2, R_PAD, D), jnp.float32),
          jax.ShapeDtypeStruct((N_TILES, R_PAD), jnp.float32),
      ],
      mesh=mesh,
      compiler_params=pltpu.CompilerParams(needs_layout_passes=False),
      scratch_types=[
          pltpu.VMEM((CHUNK, EDGE_BLK), jnp.int32),
          pltpu.VMEM((CHUNK, EDGE_BLK), jnp.int32),
          pltpu.VMEM((EDGE_BLK, D), jnp.float32),
          pltpu.VMEM((EDGE_BLK, D), jnp.float32),
          pltpu.VMEM((R_PAD,), jnp.float32),
          pltpu.VMEM_SHARED((R_PAD, D), jnp.float32),
          pltpu.SemaphoreType.DMA,
          pltpu.SemaphoreType.DMA,
      ],
  )
  def k(x_hbm, src_hbm, dst_hbm, zrow_hbm, psum_hbm, pcnt_hbm,
        srcv, dstv, r0, r1, hist, accum, s0, s1):
    cid = lax.axis_index("c")
    sid = lax.axis_index("s")
    wid = cid * 16 + sid
    base = sid * ROWS_PT

    ones_v = jnp.ones((L,), jnp.float32)
    zero_v = jnp.zeros((L,), jnp.float32)

    def init_hist(r, carry):
      hist[pl.ds(r * L, L)] = zero_v
      return carry

    lax.fori_loop(0, R_PAD // L, init_hist, 0)

    pltpu.sync_copy(zrow_hbm, accum.at[pl.ds(base, ROWS_PT)])
    plsc.subcore_barrier()

    def fire(j, buf, sem):
      pltpu.async_copy(x_hbm.at[srcv.at[j]], buf, sem)

    def wait(j, buf, sem):
      pltpu.make_async_copy(x_hbm.at[srcv.at[j]], buf, sem).wait()

    def scat(j, buf):
      pltpu.sync_copy(buf, accum.at[dstv.at[j]], add=True)
      for u in range(EDGE_BLK // L):
        idx = dstv[j, pl.ds(u * L, L)]
        plsc.addupdate_scatter(hist, [idx], ones_v)

    def chunk_body(c, carry):
      pltpu.sync_copy(src_hbm.at[wid, pl.ds(c * CHUNK, CHUNK)], srcv)
      pltpu.sync_copy(dst_hbm.at[wid, pl.ds(c * CHUNK, CHUNK)], dstv)
      fire(0, r0, s0)
      fire(1, r1, s1)

      def pipe(j2, inner):
        j = 2 * j2
        wait(j, r0, s0)
        scat(j, r0)
        fire(j + 2, r0, s0)
        wait(j + 1, r1, s1)
        scat(j + 1, r1)
        fire(j + 3, r1, s1)
        return inner

      lax.fori_loop(0, CHUNK // 2 - 1, pipe, carry)
      wait(CHUNK - 2, r0, s0)
      scat(CHUNK - 2, r0)
      wait(CHUNK - 1, r1, s1)
      scat(CHUNK - 1, r1)
      return carry

    lax.fori_loop(0, N_CHUNK, chunk_body, 0)
    plsc.subcore_barrier()
    pltpu.sync_copy(accum.at[pl.ds(base, ROWS_PT)],
                    psum_hbm.at[cid, pl.ds(base, ROWS_PT)])
    pltpu.sync_copy(hist, pcnt_hbm.at[wid])

  return k(x, srcp, dstp, zrow)


def _normalize(psum, pcnt, b):
  blk = 1000

  def body(ps_ref, pc_ref, o_ref):
    s = ps_ref[0] + ps_ref[1]
    c = jnp.sum(pc_ref[...], axis=1)
    o_ref[...] = s / jnp.maximum(c, 1.0)[:, None]

  return pl.pallas_call(
      body,
      grid=(b // blk,),
      in_specs=[
          pl.BlockSpec((2, blk, D), lambda i: (0, i, 0)),
          pl.BlockSpec((blk, N_TILES), lambda i: (i, 0)),
      ],
      out_specs=pl.BlockSpec((blk, D), lambda i: (i, 0)),
      out_shape=jax.ShapeDtypeStruct((b, D), jnp.float32),
  )(psum[:, :b], pcnt[:b])


def kernel(x, nodes, edge_index):
  b = nodes.shape[0]
  n = x.shape[0]
  e = edge_index.shape[1]
  pad = N_TILES * EPT - e
  ppt = pad // N_TILES
  pad_src = jnp.broadcast_to(
      (jnp.arange(ppt, dtype=jnp.int32) * 41) % n, (N_TILES, ppt))
  pad_dst = jnp.broadcast_to(
      b + (jnp.arange(ppt, dtype=jnp.int32) % (R_PAD - b)), (N_TILES, ppt))
  src = jnp.concatenate([edge_index[0].reshape(N_TILES, -1), pad_src], axis=1)
  dst = jnp.concatenate([edge_index[1].reshape(N_TILES, -1), pad_dst], axis=1)
  srcp = src.reshape(N_TILES, N_BLK, EDGE_BLK)
  dstp = dst.reshape(N_TILES, N_BLK, EDGE_BLK)
  zrow = jnp.zeros((ROWS_PT, D), jnp.float32)
  psum, pcnt = _sc_aggregate(x, srcp, dstp, zrow)
  return _normalize(psum, pcnt.T, b)
